# baseline (speedup 1.0000x reference)
import jax
import jax.numpy as jnp
from jax import lax
from jax.experimental import pallas as pl
from jax.experimental.pallas import tpu as pltpu
from jax.experimental.pallas import tpu_sc as plsc

VOCAB = 1000000
EMB = 64
NTOK = 16384
NPOS = 50
B_TOTAL = NTOK * NPOS

NC, NS = 2, 16
NW = NC * NS
B_PER_W = B_TOTAL // NW
CHUNK = 128
NCHUNK = B_PER_W // CHUNK
NBUF = 8
LAG = 2


def _gather_kernel(ids_hbm, table_hbm, out_hbm, idx_v, rows_v, gsems, osems):
    wid = lax.axis_index("c") * NS + lax.axis_index("s")
    base_w = wid * B_PER_W

    pltpu.sync_copy(ids_hbm.at[wid], idx_v)

    def start_gather(g, b):
        pltpu.async_copy(table_hbm.at[idx_v.at[g]], rows_v.at[b], gsems.at[b])

    def wait_gather(g, b):
        pltpu.make_async_copy(table_hbm.at[idx_v.at[g]], rows_v.at[b],
                              gsems.at[b]).wait()

    def out_slot(g):
        return out_hbm.at[pl.ds(base_w + g * CHUNK, CHUNK)]

    def start_store(g, b):
        pltpu.async_copy(rows_v.at[b], out_slot(g), osems.at[b])

    def wait_store(g, b):
        pltpu.make_async_copy(rows_v.at[b], out_slot(g), osems.at[b]).wait()

    for b in range(NBUF - LAG):
        start_gather(b, b)

    for g in range(LAG):
        b = g % NBUF
        wait_gather(g, b)
        start_store(g, b)
        start_gather(g + NBUF - LAG, (g + NBUF - LAG) % NBUF)

    def outer(k):
        for j in range(NBUF):
            g = LAG + k * NBUF + j
            b = (LAG + j) % NBUF
            wait_gather(g, b)
            start_store(g, b)
            wait_store(g - LAG, j)
            start_gather(g + NBUF - LAG, j)

    pl.loop(0, (NCHUNK - NBUF) // NBUF)(outer)

    for g in range(NCHUNK - NBUF + LAG, NCHUNK):
        b = g % NBUF
        wait_gather(g, b)
        start_store(g, b)

    for g in range(NCHUNK - NBUF, NCHUNK):
        wait_store(g, g % NBUF)


def _xpose_block(x_ref, o_ref):
    x = x_ref[...]
    at = x[:, 0:EMB].T
    bt = x[:, EMB:128].T
    o_ref[0] = jnp.stack([at, bt], axis=-1).reshape(EMB, 512)


def kernel(input_ids, weight):
    ids = input_ids.T.reshape(NW, NCHUNK, CHUNK)
    mesh = plsc.VectorSubcoreMesh(core_axis_name="c", subcore_axis_name="s")
    g = pl.kernel(
        _gather_kernel,
        mesh=mesh,
        compiler_params=pltpu.CompilerParams(use_tc_tiling_on_sc=False),
        out_type=jax.ShapeDtypeStruct((B_TOTAL, EMB), jnp.float32),
        scratch_types=[
            pltpu.VMEM((NCHUNK, CHUNK), jnp.int32),
            pltpu.VMEM((NBUF, CHUNK, EMB), jnp.float32),
            pltpu.SemaphoreType.DMA((NBUF,)),
            pltpu.SemaphoreType.DMA((NBUF,)),
        ],
    )(ids, weight)
    g2 = g.reshape(B_TOTAL // 2, 128)
    out3 = pl.pallas_call(
        _xpose_block,
        grid=(NPOS, NTOK // 512),
        in_specs=[pl.BlockSpec((256, 128), lambda k, tb: (k * 32 + tb, 0))],
        out_specs=pl.BlockSpec((1, EMB, 512), lambda k, tb: (k, 0, tb)),
        out_shape=jax.ShapeDtypeStruct((NPOS, EMB, NTOK), jnp.float32),
    )(g2)
    return jnp.transpose(out3, (2, 0, 1))


# speedup vs baseline: 6.4993x; 6.4993x over previous
import jax
import jax.numpy as jnp
from jax import lax
from jax.experimental import pallas as pl
from jax.experimental.pallas import tpu as pltpu
from jax.experimental.pallas import tpu_sc as plsc

VOCAB = 1000000
EMB = 64
NTOK = 16384
NPOS = 50
B_TOTAL = NTOK * NPOS

NC, NS = 2, 16
NW = NC * NS
B_PER_W = B_TOTAL // NW
CHUNK = 128
NCHUNK = B_PER_W // CHUNK
NBUF = 8
LAG = 2


def _gather_kernel(ids_hbm, table_hbm, out_hbm, idx_v, rows_v, gsems, osems):
    wid = lax.axis_index("c") * NS + lax.axis_index("s")
    base_w = wid * B_PER_W

    pltpu.sync_copy(ids_hbm.at[wid], idx_v)

    def start_gather(g, b):
        pltpu.async_copy(table_hbm.at[idx_v.at[g]], rows_v.at[b], gsems.at[b])

    def wait_gather(g, b):
        pltpu.make_async_copy(table_hbm.at[idx_v.at[g]], rows_v.at[b],
                              gsems.at[b]).wait()

    def out_slot(g):
        return out_hbm.at[pl.ds(base_w + g * CHUNK, CHUNK)]

    def start_store(g, b):
        pltpu.async_copy(rows_v.at[b], out_slot(g), osems.at[b])

    def wait_store(g, b):
        pltpu.make_async_copy(rows_v.at[b], out_slot(g), osems.at[b]).wait()

    for b in range(NBUF - LAG):
        start_gather(b, b)

    for g in range(LAG):
        b = g % NBUF
        wait_gather(g, b)
        start_store(g, b)
        start_gather(g + NBUF - LAG, (g + NBUF - LAG) % NBUF)

    def outer(k):
        for j in range(NBUF):
            g = LAG + k * NBUF + j
            b = (LAG + j) % NBUF
            wait_gather(g, b)
            start_store(g, b)
            wait_store(g - LAG, j)
            start_gather(g + NBUF - LAG, j)

    pl.loop(0, (NCHUNK - NBUF) // NBUF)(outer)

    for g in range(NCHUNK - NBUF + LAG, NCHUNK):
        b = g % NBUF
        wait_gather(g, b)
        start_store(g, b)

    for g in range(NCHUNK - NBUF, NCHUNK):
        wait_store(g, g % NBUF)


def _xpose_block(x_ref, o_ref):
    y = x_ref[...].T
    o_ref[0, :, 0:256] = y[0:EMB]
    o_ref[0, :, 256:512] = y[EMB:128]


def kernel(input_ids, weight):
    ids = (input_ids.T.reshape(NPOS, 32, 2, 256)
           .transpose(0, 1, 3, 2).reshape(NW, NCHUNK, CHUNK))
    mesh = plsc.VectorSubcoreMesh(core_axis_name="c", subcore_axis_name="s")
    g = pl.kernel(
        _gather_kernel,
        mesh=mesh,
        compiler_params=pltpu.CompilerParams(use_tc_tiling_on_sc=False),
        out_type=jax.ShapeDtypeStruct((B_TOTAL, EMB), jnp.float32),
        scratch_types=[
            pltpu.VMEM((NCHUNK, CHUNK), jnp.int32),
            pltpu.VMEM((NBUF, CHUNK, EMB), jnp.float32),
            pltpu.SemaphoreType.DMA((NBUF,)),
            pltpu.SemaphoreType.DMA((NBUF,)),
        ],
    )(ids, weight)
    g2 = g.reshape(B_TOTAL // 2, 128)
    out3 = pl.pallas_call(
        _xpose_block,
        grid=(NPOS, NTOK // 512),
        in_specs=[pl.BlockSpec((256, 128), lambda k, tb: (k * 32 + tb, 0))],
        out_specs=pl.BlockSpec((1, EMB, 512), lambda k, tb: (k, 0, tb)),
        out_shape=jax.ShapeDtypeStruct((NPOS, EMB, NTOK), jnp.float32),
    )(g2)
    return jnp.transpose(out3, (2, 0, 1))


# speedup vs baseline: 9.6815x; 1.4896x over previous
import jax
import jax.numpy as jnp
from jax import lax
from jax.experimental import pallas as pl
from jax.experimental.pallas import tpu as pltpu
from jax.experimental.pallas import tpu_sc as plsc

VOCAB = 1000000
EMB = 64
NTOK = 16384
NPOS = 50
B_TOTAL = NTOK * NPOS

NC, NS = 2, 16
NW = NC * NS
B_PER_W = B_TOTAL // NW
CHUNK = 128
NCHUNK = B_PER_W // CHUNK
NBUF = 8
LAG = 2


def _gather_kernel(ids_hbm, table_hbm, out_hbm, idx_v, rows_v, gsems, osems):
    wid = lax.axis_index("c") * NS + lax.axis_index("s")
    base_w = wid * B_PER_W

    pltpu.sync_copy(ids_hbm.at[wid], idx_v)

    def start_gather(g, b):
        pltpu.async_copy(table_hbm.at[idx_v.at[g]], rows_v.at[b], gsems.at[b])

    def wait_gather(g, b):
        pltpu.make_async_copy(table_hbm.at[idx_v.at[g]], rows_v.at[b],
                              gsems.at[b]).wait()

    def out_slot(g):
        return out_hbm.at[pl.ds(base_w + g * CHUNK, CHUNK)]

    def start_store(g, b):
        pltpu.async_copy(rows_v.at[b], out_slot(g), osems.at[b])

    def wait_store(g, b):
        pltpu.make_async_copy(rows_v.at[b], out_slot(g), osems.at[b]).wait()

    for b in range(NBUF - LAG):
        start_gather(b, b)

    for g in range(LAG):
        b = g % NBUF
        wait_gather(g, b)
        start_store(g, b)
        start_gather(g + NBUF - LAG, (g + NBUF - LAG) % NBUF)

    def outer(k):
        for j in range(NBUF):
            g = LAG + k * NBUF + j
            b = (LAG + j) % NBUF
            wait_gather(g, b)
            start_store(g, b)
            wait_store(g - LAG, j)
            start_gather(g + NBUF - LAG, j)

    pl.loop(0, (NCHUNK - NBUF) // NBUF)(outer)

    for g in range(NCHUNK - NBUF + LAG, NCHUNK):
        b = g % NBUF
        wait_gather(g, b)
        start_store(g, b)

    for g in range(NCHUNK - NBUF, NCHUNK):
        wait_store(g, g % NBUF)


def _xpose_block(x_ref, o_ref):
    y = x_ref[...].T
    o_ref[0, :, 0:1024] = y[0:EMB]
    o_ref[0, :, 1024:2048] = y[EMB:128]


def kernel(input_ids, weight):
    ids = (input_ids.T.reshape(NPOS, 8, 2, 1024)
           .transpose(0, 1, 3, 2).reshape(NW, NCHUNK, CHUNK))
    mesh = plsc.VectorSubcoreMesh(core_axis_name="c", subcore_axis_name="s")
    g = pl.kernel(
        _gather_kernel,
        mesh=mesh,
        compiler_params=pltpu.CompilerParams(use_tc_tiling_on_sc=False),
        out_type=jax.ShapeDtypeStruct((B_TOTAL, EMB), jnp.float32),
        scratch_types=[
            pltpu.VMEM((NCHUNK, CHUNK), jnp.int32),
            pltpu.VMEM((NBUF, CHUNK, EMB), jnp.float32),
            pltpu.SemaphoreType.DMA((NBUF,)),
            pltpu.SemaphoreType.DMA((NBUF,)),
        ],
    )(ids, weight)
    g2 = g.reshape(B_TOTAL // 2, 128)
    out3 = pl.pallas_call(
        _xpose_block,
        grid=(NPOS, NTOK // 2048),
        in_specs=[pl.BlockSpec((1024, 128), lambda k, tb: (k * 8 + tb, 0))],
        out_specs=pl.BlockSpec((1, EMB, 2048), lambda k, tb: (k, 0, tb)),
        out_shape=jax.ShapeDtypeStruct((NPOS, EMB, NTOK), jnp.float32),
    )(g2)
    return jnp.transpose(out3, (2, 0, 1))


# speedup vs baseline: 10.4417x; 1.0785x over previous
import jax
import jax.numpy as jnp
from jax import lax
from jax.experimental import pallas as pl
from jax.experimental.pallas import tpu as pltpu
from jax.experimental.pallas import tpu_sc as plsc

VOCAB = 1000000
EMB = 64
NTOK = 16384
NPOS = 50
B_TOTAL = NTOK * NPOS

NC, NS = 2, 16
NW = NC * NS
B_PER_W = B_TOTAL // NW
CHUNK = 128
NCHUNK = B_PER_W // CHUNK
NBUF = 8
LAG = 2


def _gather_kernel(ids_hbm, table_hbm, out_hbm, idx_v, rows_v, gsems, osems):
    wid = lax.axis_index("c") * NS + lax.axis_index("s")
    base_w = wid * B_PER_W

    pltpu.sync_copy(ids_hbm.at[wid], idx_v)

    def start_gather(g, b):
        pltpu.async_copy(table_hbm.at[idx_v.at[g]], rows_v.at[b], gsems.at[b])

    def wait_gather(g, b):
        pltpu.make_async_copy(table_hbm.at[idx_v.at[g]], rows_v.at[b],
                              gsems.at[b]).wait()

    def out_slot(g):
        return out_hbm.at[pl.ds(base_w + g * CHUNK, CHUNK)]

    def start_store(g, b):
        pltpu.async_copy(rows_v.at[b], out_slot(g), osems.at[b])

    def wait_store(g, b):
        pltpu.make_async_copy(rows_v.at[b], out_slot(g), osems.at[b]).wait()

    for b in range(NBUF - LAG):
        start_gather(b, b)

    for g in range(LAG):
        b = g % NBUF
        wait_gather(g, b)
        start_store(g, b)
        start_gather(g + NBUF - LAG, (g + NBUF - LAG) % NBUF)

    def outer(k):
        for j in range(NBUF):
            g = LAG + k * NBUF + j
            b = (LAG + j) % NBUF
            wait_gather(g, b)
            start_store(g, b)
            wait_store(g - LAG, j)
            start_gather(g + NBUF - LAG, j)

    pl.loop(0, (NCHUNK - NBUF) // NBUF)(outer)

    for g in range(NCHUNK - NBUF + LAG, NCHUNK):
        b = g % NBUF
        wait_gather(g, b)
        start_store(g, b)

    for g in range(NCHUNK - NBUF, NCHUNK):
        wait_store(g, g % NBUF)


def _xpose_block(x_ref, o_ref):
    y = x_ref[...].T
    o_ref[0, :, 0:2048] = y[0:EMB]
    o_ref[0, :, 2048:4096] = y[EMB:128]


def kernel(input_ids, weight):
    ids = (input_ids.T.reshape(NPOS, 4, 2, 2048)
           .transpose(0, 1, 3, 2).reshape(NW, NCHUNK, CHUNK))
    mesh = plsc.VectorSubcoreMesh(core_axis_name="c", subcore_axis_name="s")
    g = pl.kernel(
        _gather_kernel,
        mesh=mesh,
        compiler_params=pltpu.CompilerParams(use_tc_tiling_on_sc=False),
        out_type=jax.ShapeDtypeStruct((B_TOTAL, EMB), jnp.float32),
        scratch_types=[
            pltpu.VMEM((NCHUNK, CHUNK), jnp.int32),
            pltpu.VMEM((NBUF, CHUNK, EMB), jnp.float32),
            pltpu.SemaphoreType.DMA((NBUF,)),
            pltpu.SemaphoreType.DMA((NBUF,)),
        ],
    )(ids, weight)
    g2 = g.reshape(B_TOTAL // 2, 128)
    out3 = pl.pallas_call(
        _xpose_block,
        grid=(NPOS, NTOK // 4096),
        in_specs=[pl.BlockSpec((2048, 128), lambda k, tb: (k * 4 + tb, 0))],
        out_specs=pl.BlockSpec((1, EMB, 4096), lambda k, tb: (k, 0, tb)),
        out_shape=jax.ShapeDtypeStruct((NPOS, EMB, NTOK), jnp.float32),
    )(g2)
    return jnp.transpose(out3, (2, 0, 1))


# speedup vs baseline: 11.0175x; 1.0551x over previous
import jax
import jax.numpy as jnp
from jax import lax
from jax.experimental import pallas as pl
from jax.experimental.pallas import tpu as pltpu
from jax.experimental.pallas import tpu_sc as plsc

VOCAB = 1000000
EMB = 64
NTOK = 16384
NPOS = 50
B_TOTAL = NTOK * NPOS

NC, NS = 2, 16
NW = NC * NS
B_PER_W = B_TOTAL // NW
CHUNK = 128
NCHUNK = B_PER_W // CHUNK
NBUF = 8
LAG = 2


def _gather_kernel(ids_hbm, table_hbm, out_hbm, idx_v, rows_v, gsems, osems):
    wid = lax.axis_index("c") * NS + lax.axis_index("s")
    base_w = wid * B_PER_W

    pltpu.sync_copy(ids_hbm.at[wid], idx_v)

    def start_gather(g, b):
        pltpu.async_copy(table_hbm.at[idx_v.at[g]], rows_v.at[b], gsems.at[b])

    def wait_gather(g, b):
        pltpu.make_async_copy(table_hbm.at[idx_v.at[g]], rows_v.at[b],
                              gsems.at[b]).wait()

    def out_slot(g):
        return out_hbm.at[pl.ds(base_w + g * CHUNK, CHUNK)]

    def start_store(g, b):
        pltpu.async_copy(rows_v.at[b], out_slot(g), osems.at[b])

    def wait_store(g, b):
        pltpu.make_async_copy(rows_v.at[b], out_slot(g), osems.at[b]).wait()

    for b in range(NBUF - LAG):
        start_gather(b, b)

    for g in range(LAG):
        b = g % NBUF
        wait_gather(g, b)
        start_store(g, b)
        start_gather(g + NBUF - LAG, (g + NBUF - LAG) % NBUF)

    def outer(k):
        for j in range(NBUF):
            g = LAG + k * NBUF + j
            b = (LAG + j) % NBUF
            wait_gather(g, b)
            start_store(g, b)
            wait_store(g - LAG, j)
            start_gather(g + NBUF - LAG, j)

    pl.loop(0, (NCHUNK - NBUF) // NBUF)(outer)

    for g in range(NCHUNK - NBUF + LAG, NCHUNK):
        b = g % NBUF
        wait_gather(g, b)
        start_store(g, b)

    for g in range(NCHUNK - NBUF, NCHUNK):
        wait_store(g, g % NBUF)


def _xpose_block(x_ref, o_ref):
    y = x_ref[...].T
    o_ref[0, :, 0:4096] = y[0:EMB]
    o_ref[0, :, 4096:8192] = y[EMB:128]


def kernel(input_ids, weight):
    ids = (input_ids.T.reshape(NPOS, 2, 2, 4096)
           .transpose(0, 1, 3, 2).reshape(NW, NCHUNK, CHUNK))
    mesh = plsc.VectorSubcoreMesh(core_axis_name="c", subcore_axis_name="s")
    g = pl.kernel(
        _gather_kernel,
        mesh=mesh,
        compiler_params=pltpu.CompilerParams(use_tc_tiling_on_sc=False),
        out_type=jax.ShapeDtypeStruct((B_TOTAL, EMB), jnp.float32),
        scratch_types=[
            pltpu.VMEM((NCHUNK, CHUNK), jnp.int32),
            pltpu.VMEM((NBUF, CHUNK, EMB), jnp.float32),
            pltpu.SemaphoreType.DMA((NBUF,)),
            pltpu.SemaphoreType.DMA((NBUF,)),
        ],
    )(ids, weight)
    g2 = g.reshape(B_TOTAL // 2, 128)
    out3 = pl.pallas_call(
        _xpose_block,
        grid=(NPOS, NTOK // 8192),
        in_specs=[pl.BlockSpec((4096, 128), lambda k, tb: (k * 2 + tb, 0))],
        out_specs=pl.BlockSpec((1, EMB, 8192), lambda k, tb: (k, 0, tb)),
        out_shape=jax.ShapeDtypeStruct((NPOS, EMB, NTOK), jnp.float32),
    )(g2)
    return jnp.transpose(out3, (2, 0, 1))


# speedup vs baseline: 11.2852x; 1.0243x over previous
import jax
import jax.numpy as jnp
from jax import lax
from jax.experimental import pallas as pl
from jax.experimental.pallas import tpu as pltpu
from jax.experimental.pallas import tpu_sc as plsc

VOCAB = 1000000
EMB = 64
NTOK = 16384
NPOS = 50
B_TOTAL = NTOK * NPOS

NC, NS = 2, 16
NW = NC * NS
B_PER_W = B_TOTAL // NW
CHUNK = 128
NCHUNK = B_PER_W // CHUNK
NBUF = 8
LAG = 2


def _gather_kernel(ids_hbm, table_hbm, out_hbm, idx_v, rows_v, gsems, osems):
    wid = lax.axis_index("c") * NS + lax.axis_index("s")
    base_w = wid * B_PER_W

    pltpu.sync_copy(ids_hbm.at[wid], idx_v)

    def start_gather(g, b):
        pltpu.async_copy(table_hbm.at[idx_v.at[g]], rows_v.at[b], gsems.at[b])

    def wait_gather(g, b):
        pltpu.make_async_copy(table_hbm.at[idx_v.at[g]], rows_v.at[b],
                              gsems.at[b]).wait()

    def out_slot(g):
        return out_hbm.at[pl.ds(base_w + g * CHUNK, CHUNK)]

    def start_store(g, b):
        pltpu.async_copy(rows_v.at[b], out_slot(g), osems.at[b])

    def wait_store(g, b):
        pltpu.make_async_copy(rows_v.at[b], out_slot(g), osems.at[b]).wait()

    for b in range(NBUF - LAG):
        start_gather(b, b)

    for g in range(LAG):
        b = g % NBUF
        wait_gather(g, b)
        start_store(g, b)
        start_gather(g + NBUF - LAG, (g + NBUF - LAG) % NBUF)

    def outer(k):
        for j in range(NBUF):
            g = LAG + k * NBUF + j
            b = (LAG + j) % NBUF
            wait_gather(g, b)
            start_store(g, b)
            wait_store(g - LAG, j)
            start_gather(g + NBUF - LAG, j)

    pl.loop(0, (NCHUNK - NBUF) // NBUF)(outer)

    for g in range(NCHUNK - NBUF + LAG, NCHUNK):
        b = g % NBUF
        wait_gather(g, b)
        start_store(g, b)

    for g in range(NCHUNK - NBUF, NCHUNK):
        wait_store(g, g % NBUF)


def _xpose_block(x_ref, o_ref):
    y = x_ref[...].T
    o_ref[0, :, 0:8192] = y[0:EMB]
    o_ref[0, :, 8192:16384] = y[EMB:128]


def kernel(input_ids, weight):
    ids = (input_ids.T.reshape(NPOS, 1, 2, 8192)
           .transpose(0, 1, 3, 2).reshape(NW, NCHUNK, CHUNK))
    mesh = plsc.VectorSubcoreMesh(core_axis_name="c", subcore_axis_name="s")
    g = pl.kernel(
        _gather_kernel,
        mesh=mesh,
        compiler_params=pltpu.CompilerParams(use_tc_tiling_on_sc=False),
        out_type=jax.ShapeDtypeStruct((B_TOTAL, EMB), jnp.float32),
        scratch_types=[
            pltpu.VMEM((NCHUNK, CHUNK), jnp.int32),
            pltpu.VMEM((NBUF, CHUNK, EMB), jnp.float32),
            pltpu.SemaphoreType.DMA((NBUF,)),
            pltpu.SemaphoreType.DMA((NBUF,)),
        ],
    )(ids, weight)
    g2 = g.reshape(B_TOTAL // 2, 128)
    out3 = pl.pallas_call(
        _xpose_block,
        grid=(NPOS,),
        in_specs=[pl.BlockSpec((8192, 128), lambda k: (k, 0))],
        out_specs=pl.BlockSpec((1, EMB, 16384), lambda k: (k, 0, 0)),
        out_shape=jax.ShapeDtypeStruct((NPOS, EMB, NTOK), jnp.float32),
    )(g2)
    return jnp.transpose(out3, (2, 0, 1))
